# baseline (speedup 1.0000x reference)
import functools

import jax
import jax.numpy as jnp
from jax import lax
from jax.experimental import pallas as pl
from jax.experimental.pallas import tpu as pltpu

B, D, E, K = 4096, 1024, 8, 2
BM = 512
G = B // BM
S = G + E - 1


def _moe_block(x_hbm, w_hbm, b_ref, gw_ref, gb_ref, out_hbm,
               acc_ref, xbf_ref, wmat_ref, xtmp_ref, wtmp_ref, wbf_ref,
               sx, sw, so):
    s = pl.program_id(0)

    @pl.when(s == 0)
    def _prologue():
        pltpu.make_async_copy(w_hbm.at[0], wtmp_ref.at[0], sw.at[0]).start()
        pltpu.make_async_copy(
            x_hbm.at[pl.ds(0, BM)], xtmp_ref.at[0], sx.at[0]).start()

    @pl.when(s < E)
    def _w_arrive():
        ss = s % 2
        pltpu.make_async_copy(w_hbm.at[s], wtmp_ref.at[ss], sw.at[ss]).wait()
        wbf_ref[pl.ds(s, 1)] = wtmp_ref[ss].astype(jnp.bfloat16)[None]

        @pl.when(s + 1 < E)
        def _w_prefetch():
            ns = (s + 1) % 2
            pltpu.make_async_copy(
                w_hbm.at[s + 1], wtmp_ref.at[ns], sw.at[ns]).start()

    @pl.when(s < G)
    def _x_arrive():
        ss = s % 2
        pltpu.make_async_copy(
            x_hbm.at[pl.ds(s * BM, BM)], xtmp_ref.at[ss], sx.at[ss]).wait()
        x32 = xtmp_ref[ss]

        @pl.when(s + 1 < G)
        def _x_prefetch():
            ns = (s + 1) % 2
            pltpu.make_async_copy(
                x_hbm.at[pl.ds((s + 1) * BM, BM)],
                xtmp_ref.at[ns], sx.at[ns]).start()

        logits = lax.dot_general(
            x32, gw_ref[...], (((1,), (1,)), ((), ())),
            preferred_element_type=jnp.float32) + gb_ref[...]
        cols = lax.broadcasted_iota(jnp.int32, logits.shape, 1)
        idx1 = jnp.argmax(logits, axis=1, keepdims=True)
        v1 = jnp.max(logits, axis=1, keepdims=True)
        l2 = jnp.where(cols == idx1, -jnp.inf, logits)
        idx2 = jnp.argmax(l2, axis=1, keepdims=True)
        v2 = jnp.max(l2, axis=1, keepdims=True)
        w1 = 1.0 / (1.0 + jnp.exp(v2 - v1))
        w_mat = jnp.where(cols == idx1, w1,
                          jnp.where(cols == idx2, 1.0 - w1, 0.0))
        wmat_ref[pl.ds(s * BM, BM), :] = w_mat
        xbf_ref[pl.ds(s * BM, BM), :] = x32.astype(jnp.bfloat16)
        acc_ref[pl.ds(s * BM, BM), :] = lax.dot_general(
            w_mat, b_ref[...], (((1,), (0,)), ((), ())),
            preferred_element_type=jnp.float32)

    for g in range(G):
        @pl.when(jnp.logical_and(s >= g, s < g + E))
        def _dot(g=g):
            e = s - g
            xbf = xbf_ref[g * BM:(g + 1) * BM, :]
            wv = wbf_ref[e]
            y = lax.dot_general(
                xbf, wv, (((1,), (1,)), ((), ())),
                preferred_element_type=jnp.float32)
            wm = wmat_ref[g * BM:(g + 1) * BM, :]
            ecols = lax.broadcasted_iota(jnp.int32, wm.shape, 1)
            w_col = jnp.sum(jnp.where(ecols == e, wm, 0.0),
                            axis=1, keepdims=True)
            acc_ref[g * BM:(g + 1) * BM, :] += w_col * y

    @pl.when(s >= E - 1)
    def _writeback():
        gdone = s - (E - 1)
        pltpu.make_async_copy(
            acc_ref.at[pl.ds(gdone * BM, BM)],
            out_hbm.at[pl.ds(gdone * BM, BM)], so.at[gdone % 2]).start()

        @pl.when(s >= E)
        def _drain_prev():
            pltpu.make_async_copy(
                acc_ref.at[pl.ds((gdone - 1) * BM, BM)],
                out_hbm.at[pl.ds((gdone - 1) * BM, BM)],
                so.at[(gdone - 1) % 2]).wait()

        @pl.when(s == S - 1)
        def _drain_last():
            pltpu.make_async_copy(
                acc_ref.at[pl.ds(gdone * BM, BM)],
                out_hbm.at[pl.ds(gdone * BM, BM)], so.at[gdone % 2]).wait()


@functools.partial(jax.jit)
def _moe(x, W, b, gate_W, gate_b):
    return pl.pallas_call(
        _moe_block,
        grid=(S,),
        in_specs=[
            pl.BlockSpec(memory_space=pl.ANY),
            pl.BlockSpec(memory_space=pl.ANY),
            pl.BlockSpec((E, D), lambda s: (0, 0)),
            pl.BlockSpec((E, D), lambda s: (0, 0)),
            pl.BlockSpec((1, E), lambda s: (0, 0)),
        ],
        out_specs=pl.BlockSpec(memory_space=pl.ANY),
        out_shape=jax.ShapeDtypeStruct((B, D), jnp.float32),
        scratch_shapes=[
            pltpu.VMEM((B, D), jnp.float32),
            pltpu.VMEM((B, D), jnp.bfloat16),
            pltpu.VMEM((B, E), jnp.float32),
            pltpu.VMEM((2, BM, D), jnp.float32),
            pltpu.VMEM((2, D, D), jnp.float32),
            pltpu.VMEM((E, D, D), jnp.bfloat16),
            pltpu.SemaphoreType.DMA((2,)),
            pltpu.SemaphoreType.DMA((2,)),
            pltpu.SemaphoreType.DMA((2,)),
        ],
        compiler_params=pltpu.CompilerParams(
            dimension_semantics=("arbitrary",),
        ),
    )(x, W, b, gate_W, gate_b.reshape(1, E))


def kernel(x, W, b, gate_W, gate_b):
    return _moe(x, W, b, gate_W, gate_b)


# speedup vs baseline: 1.2506x; 1.2506x over previous
import functools

import jax
import jax.numpy as jnp
from jax import lax
from jax.experimental import pallas as pl
from jax.experimental.pallas import tpu as pltpu

B, D, E, K = 4096, 1024, 8, 2
BM = 512
G = B // BM


def _moe_block(x_hbm, w_hbm, b_ref, gw_ref, gb_ref, out_hbm,
               acc_ref, xbf_ref, wmat_ref, xtmp_ref, wtmp_ref, wbf_ref,
               sx, sw, so):
    e = pl.program_id(0)
    es = e % 2

    @pl.when(e == 0)
    def _w_prologue():
        pltpu.make_async_copy(w_hbm.at[0], wtmp_ref.at[0], sw.at[0]).start()

    pltpu.make_async_copy(w_hbm.at[e], wtmp_ref.at[es], sw.at[es]).wait()
    wbf_ref[...] = wtmp_ref[es].astype(jnp.bfloat16)

    @pl.when(e + 1 < E)
    def _w_prefetch():
        ns = (e + 1) % 2
        pltpu.make_async_copy(
            w_hbm.at[e + 1], wtmp_ref.at[ns], sw.at[ns]).start()

    @pl.when(e == 0)
    def _first_pass():
        for g0 in range(3):
            pltpu.make_async_copy(
                x_hbm.at[pl.ds(g0 * BM, BM)],
                xtmp_ref.at[g0], sx.at[g0]).start()
        for g in range(G):
            if g + 3 < G:
                pltpu.make_async_copy(
                    x_hbm.at[pl.ds((g + 3) * BM, BM)],
                    xtmp_ref.at[(g + 3) % 4], sx.at[(g + 3) % 4]).start()
            pltpu.make_async_copy(
                x_hbm.at[pl.ds(g * BM, BM)],
                xtmp_ref.at[g % 4], sx.at[g % 4]).wait()
            x32 = xtmp_ref[g % 4]

            logits = lax.dot_general(
                x32, gw_ref[...], (((1,), (1,)), ((), ())),
                preferred_element_type=jnp.float32) + gb_ref[...]
            cols = lax.broadcasted_iota(jnp.int32, logits.shape, 1)
            idx1 = jnp.argmax(logits, axis=1, keepdims=True)
            v1 = jnp.max(logits, axis=1, keepdims=True)
            l2 = jnp.where(cols == idx1, -jnp.inf, logits)
            idx2 = jnp.argmax(l2, axis=1, keepdims=True)
            v2 = jnp.max(l2, axis=1, keepdims=True)
            w1 = 1.0 / (1.0 + jnp.exp(v2 - v1))
            w_mat = jnp.where(cols == idx1, w1,
                              jnp.where(cols == idx2, 1.0 - w1, 0.0))
            wmat_ref[g * BM:(g + 1) * BM, :] = w_mat

            xbf = x32.astype(jnp.bfloat16)
            xbf_ref[g * BM:(g + 1) * BM, :] = xbf

            acc = lax.dot_general(
                w_mat, b_ref[...], (((1,), (0,)), ((), ())),
                preferred_element_type=jnp.float32)
            y = lax.dot_general(
                xbf, wbf_ref[...], (((1,), (1,)), ((), ())),
                preferred_element_type=jnp.float32)
            acc_ref[g * BM:(g + 1) * BM, :] = acc + w_mat[:, 0:1] * y

    @pl.when(e != 0)
    def _accumulate():
        for g in range(G):
            xbf = xbf_ref[g * BM:(g + 1) * BM, :]
            y = lax.dot_general(
                xbf, wbf_ref[...], (((1,), (1,)), ((), ())),
                preferred_element_type=jnp.float32)
            wm = wmat_ref[g * BM:(g + 1) * BM, :]
            ecols = lax.broadcasted_iota(jnp.int32, wm.shape, 1)
            w_col = jnp.sum(jnp.where(ecols == e, wm, 0.0),
                            axis=1, keepdims=True)
            acc_ref[g * BM:(g + 1) * BM, :] += w_col * y

            @pl.when(e == E - 1)
            def _fire(g=g):
                pltpu.make_async_copy(
                    acc_ref.at[pl.ds(g * BM, BM)],
                    out_hbm.at[pl.ds(g * BM, BM)], so).start()

    @pl.when(e == E - 1)
    def _writeback():
        for g in range(G):
            pltpu.make_async_copy(
                acc_ref.at[pl.ds(g * BM, BM)],
                out_hbm.at[pl.ds(g * BM, BM)], so).wait()


@functools.partial(jax.jit)
def _moe(x, W, b, gate_W, gate_b):
    return pl.pallas_call(
        _moe_block,
        grid=(E,),
        in_specs=[
            pl.BlockSpec(memory_space=pl.ANY),
            pl.BlockSpec(memory_space=pl.ANY),
            pl.BlockSpec((E, D), lambda e: (0, 0)),
            pl.BlockSpec((E, D), lambda e: (0, 0)),
            pl.BlockSpec((1, E), lambda e: (0, 0)),
        ],
        out_specs=pl.BlockSpec(memory_space=pl.ANY),
        out_shape=jax.ShapeDtypeStruct((B, D), jnp.float32),
        scratch_shapes=[
            pltpu.VMEM((B, D), jnp.float32),
            pltpu.VMEM((B, D), jnp.bfloat16),
            pltpu.VMEM((B, E), jnp.float32),
            pltpu.VMEM((4, BM, D), jnp.float32),
            pltpu.VMEM((2, D, D), jnp.float32),
            pltpu.VMEM((D, D), jnp.bfloat16),
            pltpu.SemaphoreType.DMA((4,)),
            pltpu.SemaphoreType.DMA((2,)),
            pltpu.SemaphoreType.DMA,
        ],
        compiler_params=pltpu.CompilerParams(
            dimension_semantics=("arbitrary",),
        ),
    )(x, W, b, gate_W, gate_b.reshape(1, E))


def kernel(x, W, b, gate_W, gate_b):
    return _moe(x, W, b, gate_W, gate_b)
